# baseline (speedup 1.0000x reference)
import jax
import jax.numpy as jnp
from jax import lax
from jax.experimental import pallas as pl
from jax.experimental.pallas import tpu as pltpu
from jax.experimental.pallas import tpu_sc as plsc

N = 10000
E = 320000
D = 128
NC = 2
NS = 16
L = 16
NW = NC * NS
ER = E // D
RPT = 80
ERP = NW * RPT
NP = 10240
NPT = NP // NS
DEGN = 10240
DPT = DEGN // NS
CH = 4

_DNUMS = lax.GatherDimensionNumbers(offset_dims=(), collapsed_slice_dims=(0,),
                                    start_index_map=(0,))


def _make_sc_aggregate(with_deg):
    def body(*refs):
        if with_deg:
            (x_hbm, src_hbm, dst_hbm, ew_hbm, outp, outdeg,
             acc, degacc, src_st, dst_st, ew_st, rows2, ones_v, zdeg,
             g0, g1, s0, s1, stsem, dsem) = refs
        else:
            (x_hbm, src_hbm, dst_hbm, ew_hbm, outp,
             acc, src_st, dst_st, ew_st, rows2,
             g0, g1, s0, s1, stsem) = refs
            dsem = None
        gsems = (g0, g1)
        ssems = (s0, s1)
        c = lax.axis_index("c")
        s = lax.axis_index("s")
        w = c * NS + s

        def _zrows(i, _):
            for g in range(D // L):
                rows2[0, i, pl.ds(g * L, L)] = jnp.zeros((L,), jnp.float32)
            return 0
        lax.fori_loop(0, D, _zrows, 0)
        for j in range(NPT // D):
            pltpu.sync_copy(rows2.at[0], acc.at[pl.ds(s * NPT + j * D, D)])

        if with_deg:
            for g in range(D // L):
                ones_v[pl.ds(g * L, L)] = jnp.ones((L,), jnp.float32)
            def _zd(i, _):
                zdeg[pl.ds(i * L, L)] = jnp.zeros((L,), jnp.float32)
                return 0
            lax.fori_loop(0, DPT // L, _zd, 0)
            pltpu.sync_copy(zdeg, degacc.at[pl.ds(s * DPT, DPT)])

        lo = w * RPT
        n_rows = jnp.minimum(ER - lo, RPT)
        n_chunks = n_rows // CH

        pltpu.sync_copy(src_hbm.at[pl.ds(lo, CH)], src_st.at[pl.ds(0, CH)])
        pltpu.sync_copy(dst_hbm.at[pl.ds(lo, CH)], dst_st.at[pl.ds(0, CH)])
        pltpu.sync_copy(ew_hbm.at[pl.ds(lo, CH)], ew_st.at[pl.ds(0, CH)])

        pltpu.async_copy(x_hbm.at[src_st.at[0]], rows2.at[0], gsems[0])
        pltpu.async_copy(x_hbm.at[src_st.at[1]], rows2.at[1], gsems[1])

        plsc.subcore_barrier()

        def _scale(b, sl, glo, ghi):
            def _grp(g16, _):
                e0 = g16 * L
                ew16 = ew_st[sl, pl.ds(e0, L)]
                for l in range(L):
                    e = e0 + l
                    wv = ew16[l]
                    for g in range(D // L):
                        sl2 = pl.ds(g * L, L)
                        rows2[b, e, sl2] = rows2[b, e, sl2] * wv
                return 0
            lax.fori_loop(glo, ghi, _grp, 0)

        def _chunk(k, _):
            base = k * CH
            off = (k % 2) * CH
            noff = CH - off
            have_next = k + 1 < n_chunks

            @pl.when(have_next)
            def _():
                nlo = lo + base + CH
                pltpu.async_copy(src_hbm.at[pl.ds(nlo, CH)],
                                 src_st.at[pl.ds(noff, CH)], stsem)
                pltpu.async_copy(dst_hbm.at[pl.ds(nlo, CH)],
                                 dst_st.at[pl.ds(noff, CH)], stsem)
                pltpu.async_copy(ew_hbm.at[pl.ds(nlo, CH)],
                                 ew_st.at[pl.ds(noff, CH)], stsem)

            for i in range(CH):
                b = i % 2
                r = base + i
                sl = off + i
                pltpu.make_async_copy(x_hbm.at[src_st.at[sl]], rows2.at[b],
                                      gsems[b]).wait()
                _scale(b, sl, 0, 4)

                def _midrow():
                    pltpu.make_async_copy(rows2.at[1 - b],
                                          acc.at[dst_st.at[sl]],
                                          ssems[1 - b]).wait()
                    if with_deg:
                        pltpu.make_async_copy(ones_v,
                                              degacc.at[dst_st.at[sl]],
                                              dsem).wait()
                    if i < CH - 1:
                        pltpu.async_copy(x_hbm.at[src_st.at[sl + 1]],
                                         rows2.at[1 - b], gsems[1 - b])
                if i == 0:
                    @pl.when(k > 0)
                    def _():
                        _midrow()
                elif i == CH - 1:
                    @pl.when(have_next)
                    def _():
                        nlo = lo + base + CH
                        pltpu.make_async_copy(
                            src_hbm.at[pl.ds(nlo, CH)],
                            src_st.at[pl.ds(noff, CH)], stsem).wait()
                        pltpu.make_async_copy(
                            dst_hbm.at[pl.ds(nlo, CH)],
                            dst_st.at[pl.ds(noff, CH)], stsem).wait()
                        pltpu.make_async_copy(
                            ew_hbm.at[pl.ds(nlo, CH)],
                            ew_st.at[pl.ds(noff, CH)], stsem).wait()
                    pltpu.make_async_copy(rows2.at[1 - b],
                                          acc.at[dst_st.at[sl]],
                                          ssems[1 - b]).wait()
                    if with_deg:
                        pltpu.make_async_copy(ones_v,
                                              degacc.at[dst_st.at[sl]],
                                              dsem).wait()
                    @pl.when(have_next)
                    def _():
                        pltpu.async_copy(x_hbm.at[src_st.at[noff]],
                                         rows2.at[1 - b], gsems[1 - b])
                else:
                    _midrow()

                _scale(b, sl, 4, 8)

                pltpu.async_copy(rows2.at[b], acc.at[dst_st.at[sl]],
                                 ssems[b], add=True)
                if with_deg:
                    pltpu.async_copy(ones_v, degacc.at[dst_st.at[sl]],
                                     dsem, add=True)
            return 0
        lax.fori_loop(0, n_chunks, _chunk, 0)

        last_off = ((n_chunks - 1) % 2) * CH
        last_sl = last_off + CH - 1
        pltpu.make_async_copy(rows2.at[1], acc.at[dst_st.at[last_sl]],
                              ssems[1]).wait()
        if with_deg:
            pltpu.make_async_copy(ones_v, degacc.at[dst_st.at[last_sl]],
                                  dsem).wait()

        plsc.subcore_barrier()

        pltpu.sync_copy(acc.at[pl.ds(s * NPT, NPT)],
                        outp.at[c, pl.ds(s * NPT, NPT)])
        if with_deg:
            pltpu.sync_copy(degacc.at[pl.ds(s * DPT, DPT)],
                            outdeg.at[c, pl.ds(s * DPT, DPT)])

    if with_deg:
        out_type = (jax.ShapeDtypeStruct((NC, NP, D), jnp.float32),
                    jax.ShapeDtypeStruct((NC, DEGN), jnp.float32))
    else:
        out_type = jax.ShapeDtypeStruct((NC, NP, D), jnp.float32)
    scratch = [
        pltpu.VMEM_SHARED((NP, D), jnp.float32),
    ]
    if with_deg:
        scratch.append(pltpu.VMEM_SHARED((DEGN,), jnp.float32))
    scratch += [
        pltpu.VMEM((2 * CH, D), jnp.int32),
        pltpu.VMEM((2 * CH, D), jnp.int32),
        pltpu.VMEM((2 * CH, D), jnp.float32),
        pltpu.VMEM((2, D, D), jnp.float32),
    ]
    if with_deg:
        scratch += [
            pltpu.VMEM((D,), jnp.float32),
            pltpu.VMEM((DPT,), jnp.float32),
        ]
    scratch += [pltpu.SemaphoreType.DMA] * 5
    if with_deg:
        scratch += [pltpu.SemaphoreType.DMA]
    return pl.kernel(
        body,
        out_type=out_type,
        mesh=plsc.VectorSubcoreMesh(core_axis_name="c", subcore_axis_name="s",
                                    num_cores=NC, num_subcores=NS),
        scratch_types=scratch,
    )


_sc_aggregate_deg = _make_sc_aggregate(True)
_sc_aggregate = _make_sc_aggregate(False)


def _tc_body(p_ref, deg_ref, x_ref, w_ref, b_ref, o_ref):
    p = p_ref[0, :N] + p_ref[1, :N]
    deg = deg_ref[...]
    degs = jnp.maximum(deg[:, 0:1] + deg[:, 1:2], 1.0)
    agg = p / degs
    x = x_ref[...]
    w = w_ref[...]
    h = (lax.dot_general(agg, w[:, :D], (((1,), (1,)), ((), ())),
                         preferred_element_type=jnp.float32)
         + lax.dot_general(x, w[:, D:], (((1,), (1,)), ((), ())),
                           preferred_element_type=jnp.float32)
         + b_ref[...])
    o_ref[...] = jnp.maximum(h, 0.0)


_tc_layer = pl.pallas_call(
    _tc_body,
    out_shape=jax.ShapeDtypeStruct((N, D), jnp.float32),
)


def kernel(n_feat, edge_index, edge_weights, W1, b1, W2, b2, W3, b3):
    src = edge_index[0].reshape(ER, D)
    dst = edge_index[1].reshape(ER, D)
    ew = edge_weights.reshape(ER, D)
    padr = ERP - ER
    src = jnp.pad(src, ((0, padr), (0, 0)))
    dst = jnp.pad(dst, ((0, padr), (0, 0)))
    ew = jnp.pad(ew, ((0, padr), (0, 0)))

    h = n_feat
    deg2 = None
    for (W, b) in ((W1, b1), (W2, b2), (W3, b3)):
        if deg2 is None:
            partial, degp = _sc_aggregate_deg(h, src, dst, ew)
            deg2 = degp[:, :N].T
        else:
            partial = _sc_aggregate(h, src, dst, ew)
        h = _tc_layer(partial, deg2, h, W, b.reshape(1, D))
    return h


# speedup vs baseline: 1.0198x; 1.0198x over previous
import jax
import jax.numpy as jnp
from jax import lax
from jax.experimental import pallas as pl
from jax.experimental.pallas import tpu as pltpu
from jax.experimental.pallas import tpu_sc as plsc

N = 10000
E = 320000
D = 128
NC = 2
NS = 16
L = 16
NW = NC * NS
ER = E // D
RPT = 80
ERP = NW * RPT
NP = 10240
NPT = NP // NS
DEGN = 10240
DPT = DEGN // NS
CH = 4

_DNUMS = lax.GatherDimensionNumbers(offset_dims=(), collapsed_slice_dims=(0,),
                                    start_index_map=(0,))


def _make_sc_aggregate(with_deg):
    def body(*refs):
        if with_deg:
            (x_hbm, src_hbm, dst_hbm, ew_hbm, outp, outdeg,
             acc, degacc, src_st, dst_st, ew_st, rows2, ones_v, zdeg,
             g0, g1, s0, s1, stsem, dsem) = refs
        else:
            (x_hbm, src_hbm, dst_hbm, ew_hbm, outp,
             acc, src_st, dst_st, ew_st, rows2,
             g0, g1, s0, s1, stsem) = refs
            dsem = None
        gsems = (g0, g1)
        ssems = (s0, s1)
        c = lax.axis_index("c")
        s = lax.axis_index("s")
        w = c * NS + s

        def _zrows(i, _):
            for g in range(D // L):
                rows2[0, i, pl.ds(g * L, L)] = jnp.zeros((L,), jnp.float32)
            return 0
        lax.fori_loop(0, D, _zrows, 0)
        for j in range(NPT // D):
            pltpu.sync_copy(rows2.at[0], acc.at[pl.ds(s * NPT + j * D, D)])

        if with_deg:
            for g in range(D // L):
                ones_v[pl.ds(g * L, L)] = jnp.ones((L,), jnp.float32)
            def _zd(i, _):
                zdeg[pl.ds(i * L, L)] = jnp.zeros((L,), jnp.float32)
                return 0
            lax.fori_loop(0, DPT // L, _zd, 0)
            pltpu.sync_copy(zdeg, degacc.at[pl.ds(s * DPT, DPT)])

        lo = w * RPT
        n_rows = jnp.minimum(ER - lo, RPT)
        n_chunks = n_rows // CH

        pltpu.sync_copy(src_hbm.at[pl.ds(lo, CH)], src_st.at[pl.ds(0, CH)])
        pltpu.sync_copy(dst_hbm.at[pl.ds(lo, CH)], dst_st.at[pl.ds(0, CH)])
        pltpu.sync_copy(ew_hbm.at[pl.ds(lo, CH)], ew_st.at[pl.ds(0, CH)])

        pltpu.async_copy(x_hbm.at[src_st.at[0]], rows2.at[0], gsems[0])
        pltpu.async_copy(x_hbm.at[src_st.at[1]], rows2.at[1], gsems[1])

        plsc.subcore_barrier()

        def _scale(b, sl, glo, ghi):
            def _grp(g16, _):
                e0 = g16 * L
                ew16 = ew_st[sl, pl.ds(e0, L)]
                for l in range(L):
                    wv = lax.gather(ew16, jnp.full((L, 1), l, jnp.int32),
                                    _DNUMS, (1,),
                                    mode=lax.GatherScatterMode.PROMISE_IN_BOUNDS)
                    for g in range(D // L):
                        sl2 = pl.ds(g * L, L)
                        rows2[b, e0 + l, sl2] = rows2[b, e0 + l, sl2] * wv
                return 0
            lax.fori_loop(glo, ghi, _grp, 0)

        def _chunk(k, _):
            base = k * CH
            off = (k % 2) * CH
            noff = CH - off
            have_next = k + 1 < n_chunks

            @pl.when(have_next)
            def _():
                nlo = lo + base + CH
                pltpu.async_copy(src_hbm.at[pl.ds(nlo, CH)],
                                 src_st.at[pl.ds(noff, CH)], stsem)
                pltpu.async_copy(dst_hbm.at[pl.ds(nlo, CH)],
                                 dst_st.at[pl.ds(noff, CH)], stsem)
                pltpu.async_copy(ew_hbm.at[pl.ds(nlo, CH)],
                                 ew_st.at[pl.ds(noff, CH)], stsem)

            for i in range(CH):
                b = i % 2
                r = base + i
                sl = off + i
                pltpu.make_async_copy(x_hbm.at[src_st.at[sl]], rows2.at[b],
                                      gsems[b]).wait()
                _scale(b, sl, 0, 4)

                def _midrow():
                    pltpu.make_async_copy(rows2.at[1 - b],
                                          acc.at[dst_st.at[sl]],
                                          ssems[1 - b]).wait()
                    if with_deg:
                        pltpu.make_async_copy(ones_v,
                                              degacc.at[dst_st.at[sl]],
                                              dsem).wait()
                    if i < CH - 1:
                        pltpu.async_copy(x_hbm.at[src_st.at[sl + 1]],
                                         rows2.at[1 - b], gsems[1 - b])
                if i == 0:
                    @pl.when(k > 0)
                    def _():
                        _midrow()
                elif i == CH - 1:
                    @pl.when(have_next)
                    def _():
                        nlo = lo + base + CH
                        pltpu.make_async_copy(
                            src_hbm.at[pl.ds(nlo, CH)],
                            src_st.at[pl.ds(noff, CH)], stsem).wait()
                        pltpu.make_async_copy(
                            dst_hbm.at[pl.ds(nlo, CH)],
                            dst_st.at[pl.ds(noff, CH)], stsem).wait()
                        pltpu.make_async_copy(
                            ew_hbm.at[pl.ds(nlo, CH)],
                            ew_st.at[pl.ds(noff, CH)], stsem).wait()
                    pltpu.make_async_copy(rows2.at[1 - b],
                                          acc.at[dst_st.at[sl]],
                                          ssems[1 - b]).wait()
                    if with_deg:
                        pltpu.make_async_copy(ones_v,
                                              degacc.at[dst_st.at[sl]],
                                              dsem).wait()
                    @pl.when(have_next)
                    def _():
                        pltpu.async_copy(x_hbm.at[src_st.at[noff]],
                                         rows2.at[1 - b], gsems[1 - b])
                else:
                    _midrow()

                _scale(b, sl, 4, 8)

                pltpu.async_copy(rows2.at[b], acc.at[dst_st.at[sl]],
                                 ssems[b], add=True)
                if with_deg:
                    pltpu.async_copy(ones_v, degacc.at[dst_st.at[sl]],
                                     dsem, add=True)
            return 0
        lax.fori_loop(0, n_chunks, _chunk, 0)

        last_off = ((n_chunks - 1) % 2) * CH
        last_sl = last_off + CH - 1
        pltpu.make_async_copy(rows2.at[1], acc.at[dst_st.at[last_sl]],
                              ssems[1]).wait()
        if with_deg:
            pltpu.make_async_copy(ones_v, degacc.at[dst_st.at[last_sl]],
                                  dsem).wait()

        plsc.subcore_barrier()

        pltpu.sync_copy(acc.at[pl.ds(s * NPT, NPT)],
                        outp.at[c, pl.ds(s * NPT, NPT)])
        if with_deg:
            pltpu.sync_copy(degacc.at[pl.ds(s * DPT, DPT)],
                            outdeg.at[c, pl.ds(s * DPT, DPT)])

    if with_deg:
        out_type = (jax.ShapeDtypeStruct((NC, NP, D), jnp.float32),
                    jax.ShapeDtypeStruct((NC, DEGN), jnp.float32))
    else:
        out_type = jax.ShapeDtypeStruct((NC, NP, D), jnp.float32)
    scratch = [
        pltpu.VMEM_SHARED((NP, D), jnp.float32),
    ]
    if with_deg:
        scratch.append(pltpu.VMEM_SHARED((DEGN,), jnp.float32))
    scratch += [
        pltpu.VMEM((2 * CH, D), jnp.int32),
        pltpu.VMEM((2 * CH, D), jnp.int32),
        pltpu.VMEM((2 * CH, D), jnp.float32),
        pltpu.VMEM((2, D, D), jnp.float32),
    ]
    if with_deg:
        scratch += [
            pltpu.VMEM((D,), jnp.float32),
            pltpu.VMEM((DPT,), jnp.float32),
        ]
    scratch += [pltpu.SemaphoreType.DMA] * 5
    if with_deg:
        scratch += [pltpu.SemaphoreType.DMA]
    return pl.kernel(
        body,
        out_type=out_type,
        mesh=plsc.VectorSubcoreMesh(core_axis_name="c", subcore_axis_name="s",
                                    num_cores=NC, num_subcores=NS),
        scratch_types=scratch,
    )


_sc_aggregate_deg = _make_sc_aggregate(True)
_sc_aggregate = _make_sc_aggregate(False)


def _tc_body(p_ref, deg_ref, x_ref, w_ref, b_ref, o_ref):
    p = p_ref[0, :N] + p_ref[1, :N]
    deg = deg_ref[...]
    degs = jnp.maximum(deg[:, 0:1] + deg[:, 1:2], 1.0)
    agg = p / degs
    x = x_ref[...]
    w = w_ref[...]
    h = (lax.dot_general(agg, w[:, :D], (((1,), (1,)), ((), ())),
                         preferred_element_type=jnp.float32)
         + lax.dot_general(x, w[:, D:], (((1,), (1,)), ((), ())),
                           preferred_element_type=jnp.float32)
         + b_ref[...])
    o_ref[...] = jnp.maximum(h, 0.0)


_tc_layer = pl.pallas_call(
    _tc_body,
    out_shape=jax.ShapeDtypeStruct((N, D), jnp.float32),
)


def kernel(n_feat, edge_index, edge_weights, W1, b1, W2, b2, W3, b3):
    src = edge_index[0].reshape(ER, D)
    dst = edge_index[1].reshape(ER, D)
    ew = edge_weights.reshape(ER, D)
    padr = ERP - ER
    src = jnp.pad(src, ((0, padr), (0, 0)))
    dst = jnp.pad(dst, ((0, padr), (0, 0)))
    ew = jnp.pad(ew, ((0, padr), (0, 0)))

    h = n_feat
    deg2 = None
    for (W, b) in ((W1, b1), (W2, b2), (W3, b3)):
        if deg2 is None:
            partial, degp = _sc_aggregate_deg(h, src, dst, ew)
            deg2 = degp[:, :N].T
        else:
            partial = _sc_aggregate(h, src, dst, ew)
        h = _tc_layer(partial, deg2, h, W, b.reshape(1, D))
    return h


# speedup vs baseline: 1.0458x; 1.0254x over previous
import jax
import jax.numpy as jnp
from jax import lax
from jax.experimental import pallas as pl
from jax.experimental.pallas import tpu as pltpu
from jax.experimental.pallas import tpu_sc as plsc

N = 10000
E = 320000
D = 128
NC = 2
NS = 16
L = 16
NW = NC * NS
ER = E // D
RPT = 80
ERP = NW * RPT
NP = 10240
NPT = NP // NS
DEGN = 10240
DPT = DEGN // NS
CH = 4

_DNUMS = lax.GatherDimensionNumbers(offset_dims=(), collapsed_slice_dims=(0,),
                                    start_index_map=(0,))


def _make_sc_aggregate(with_deg):
    def body(*refs):
        if with_deg:
            (x_hbm, src_hbm, dst_hbm, ew_hbm, outp, outdeg,
             acc, degacc, src_st, dst_st, ew_st, rows2, ones_v, zdeg,
             g0, g1, s0, s1, stsem, dsem) = refs
        else:
            (x_hbm, src_hbm, dst_hbm, ew_hbm, outp,
             acc, src_st, dst_st, ew_st, rows2,
             g0, g1, s0, s1, stsem) = refs
            dsem = None
        gsems = (g0, g1)
        ssems = (s0, s1)
        c = lax.axis_index("c")
        s = lax.axis_index("s")
        w = c * NS + s

        def _zrows(i, _):
            for g in range(D // L):
                rows2[0, i, pl.ds(g * L, L)] = jnp.zeros((L,), jnp.float32)
            return 0
        lax.fori_loop(0, D, _zrows, 0)
        for j in range(NPT // D):
            pltpu.sync_copy(rows2.at[0], acc.at[pl.ds(s * NPT + j * D, D)])

        if with_deg:
            for g in range(D // L):
                ones_v[pl.ds(g * L, L)] = jnp.ones((L,), jnp.float32)
            def _zd(i, _):
                zdeg[pl.ds(i * L, L)] = jnp.zeros((L,), jnp.float32)
                return 0
            lax.fori_loop(0, DPT // L, _zd, 0)
            pltpu.sync_copy(zdeg, degacc.at[pl.ds(s * DPT, DPT)])

        lo = w * RPT
        n_rows = jnp.minimum(ER - lo, RPT)
        n_chunks = n_rows // CH

        pltpu.sync_copy(src_hbm.at[pl.ds(lo, CH)], src_st.at[pl.ds(0, CH)])
        pltpu.sync_copy(dst_hbm.at[pl.ds(lo, CH)], dst_st.at[pl.ds(0, CH)])
        pltpu.sync_copy(ew_hbm.at[pl.ds(lo, CH)], ew_st.at[pl.ds(0, CH)])

        pltpu.async_copy(x_hbm.at[src_st.at[0]], rows2.at[0], gsems[0])
        pltpu.async_copy(x_hbm.at[src_st.at[1]], rows2.at[1], gsems[1])

        plsc.subcore_barrier()

        def _scale(b, sl, glo, ghi):
            def _grp(g16, _):
                e0 = g16 * L
                ew16 = ew_st[sl, pl.ds(e0, L)]
                for l in range(L):
                    wv = lax.gather(ew16, jnp.full((L, 1), l, jnp.int32),
                                    _DNUMS, (1,),
                                    mode=lax.GatherScatterMode.PROMISE_IN_BOUNDS)
                    for g in range(D // L):
                        sl2 = pl.ds(g * L, L)
                        rows2[b, e0 + l, sl2] = rows2[b, e0 + l, sl2] * wv
                return 0
            lax.fori_loop(glo, ghi, _grp, 0)

        def _chunk(k, _):
            base = k * CH
            off = (k % 2) * CH
            noff = CH - off
            have_next = k + 1 < n_chunks

            @pl.when(have_next)
            def _():
                nlo = lo + base + CH
                pltpu.async_copy(src_hbm.at[pl.ds(nlo, CH)],
                                 src_st.at[pl.ds(noff, CH)], stsem)
                pltpu.async_copy(dst_hbm.at[pl.ds(nlo, CH)],
                                 dst_st.at[pl.ds(noff, CH)], stsem)
                pltpu.async_copy(ew_hbm.at[pl.ds(nlo, CH)],
                                 ew_st.at[pl.ds(noff, CH)], stsem)

            for i in range(CH):
                b = i % 2
                r = base + i
                sl = off + i
                pltpu.make_async_copy(x_hbm.at[src_st.at[sl]], rows2.at[b],
                                      gsems[b]).wait()
                _scale(b, sl, 0, 4)

                def _midrow():
                    pltpu.make_async_copy(rows2.at[1 - b],
                                          acc.at[dst_st.at[sl]],
                                          ssems[1 - b]).wait()
                    if with_deg:
                        pltpu.make_async_copy(ones_v,
                                              degacc.at[dst_st.at[sl]],
                                              dsem).wait()
                    if i < CH - 1:
                        pltpu.async_copy(x_hbm.at[src_st.at[sl + 1]],
                                         rows2.at[1 - b], gsems[1 - b])
                if i == 0:
                    @pl.when(k > 0)
                    def _():
                        _midrow()
                elif i == CH - 1:
                    @pl.when(have_next)
                    def _():
                        nlo = lo + base + CH
                        pltpu.make_async_copy(
                            src_hbm.at[pl.ds(nlo, CH)],
                            src_st.at[pl.ds(noff, CH)], stsem).wait()
                        pltpu.make_async_copy(
                            dst_hbm.at[pl.ds(nlo, CH)],
                            dst_st.at[pl.ds(noff, CH)], stsem).wait()
                        pltpu.make_async_copy(
                            ew_hbm.at[pl.ds(nlo, CH)],
                            ew_st.at[pl.ds(noff, CH)], stsem).wait()
                    pltpu.make_async_copy(rows2.at[1 - b],
                                          acc.at[dst_st.at[sl]],
                                          ssems[1 - b]).wait()
                    if with_deg:
                        pltpu.make_async_copy(ones_v,
                                              degacc.at[dst_st.at[sl]],
                                              dsem).wait()
                    @pl.when(have_next)
                    def _():
                        pltpu.async_copy(x_hbm.at[src_st.at[noff]],
                                         rows2.at[1 - b], gsems[1 - b])
                else:
                    _midrow()

                _scale(b, sl, 4, 8)

                pltpu.async_copy(rows2.at[b], acc.at[dst_st.at[sl]],
                                 ssems[b], add=True)
                if with_deg:
                    pltpu.async_copy(ones_v, degacc.at[dst_st.at[sl]],
                                     dsem, add=True)
            return 0
        lax.fori_loop(0, n_chunks, _chunk, 0)

        last_off = ((n_chunks - 1) % 2) * CH
        last_sl = last_off + CH - 1
        pltpu.make_async_copy(rows2.at[1], acc.at[dst_st.at[last_sl]],
                              ssems[1]).wait()
        if with_deg:
            pltpu.make_async_copy(ones_v, degacc.at[dst_st.at[last_sl]],
                                  dsem).wait()

        plsc.subcore_barrier()

        pltpu.sync_copy(acc.at[pl.ds(s * NPT, NPT)],
                        outp.at[c, pl.ds(s * NPT, NPT)])
        if with_deg:
            pltpu.sync_copy(degacc.at[pl.ds(s * DPT, DPT)],
                            outdeg.at[c, pl.ds(s * DPT, DPT)])

    if with_deg:
        out_type = (jax.ShapeDtypeStruct((NC, NP, D), jnp.float32),
                    jax.ShapeDtypeStruct((NC, DEGN), jnp.float32))
    else:
        out_type = jax.ShapeDtypeStruct((NC, NP, D), jnp.float32)
    scratch = [
        pltpu.VMEM_SHARED((NP, D), jnp.float32),
    ]
    if with_deg:
        scratch.append(pltpu.VMEM_SHARED((DEGN,), jnp.float32))
    scratch += [
        pltpu.VMEM((2 * CH, D), jnp.int32),
        pltpu.VMEM((2 * CH, D), jnp.int32),
        pltpu.VMEM((2 * CH, D), jnp.float32),
        pltpu.VMEM((2, D, D), jnp.float32),
    ]
    if with_deg:
        scratch += [
            pltpu.VMEM((D,), jnp.float32),
            pltpu.VMEM((DPT,), jnp.float32),
        ]
    scratch += [pltpu.SemaphoreType.DMA] * 5
    if with_deg:
        scratch += [pltpu.SemaphoreType.DMA]
    return pl.kernel(
        body,
        out_type=out_type,
        mesh=plsc.VectorSubcoreMesh(core_axis_name="c", subcore_axis_name="s",
                                    num_cores=NC, num_subcores=NS),
        scratch_types=scratch,
    )


_sc_aggregate_deg = _make_sc_aggregate(True)
_sc_aggregate = _make_sc_aggregate(False)


def _tc_body(p_ref, deg_ref, x_ref, w_ref, b_ref, o_ref):
    p = p_ref[0, :N] + p_ref[1, :N]
    deg = deg_ref[0, :N] + deg_ref[1, :N]
    degs = jnp.maximum(deg, 1.0)[:, None]
    agg = p / degs
    x = x_ref[...]
    w = w_ref[...]
    h = (lax.dot_general(agg, w[:, :D], (((1,), (1,)), ((), ())),
                         preferred_element_type=jnp.float32)
         + lax.dot_general(x, w[:, D:], (((1,), (1,)), ((), ())),
                           preferred_element_type=jnp.float32)
         + b_ref[...])
    o_ref[...] = jnp.maximum(h, 0.0)


_tc_layer = pl.pallas_call(
    _tc_body,
    out_shape=jax.ShapeDtypeStruct((N, D), jnp.float32),
)


def kernel(n_feat, edge_index, edge_weights, W1, b1, W2, b2, W3, b3):
    src = edge_index[0].reshape(ER, D)
    dst = edge_index[1].reshape(ER, D)
    ew = edge_weights.reshape(ER, D)

    h = n_feat
    degp = None
    for (W, b) in ((W1, b1), (W2, b2), (W3, b3)):
        if degp is None:
            partial, degp = _sc_aggregate_deg(h, src, dst, ew)
        else:
            partial = _sc_aggregate(h, src, dst, ew)
        h = _tc_layer(partial, degp, h, W, b.reshape(1, D))
    return h
